# Initial kernel scaffold; baseline (speedup 1.0000x reference)
import jax
import jax.numpy as jnp
from jax.experimental import pallas as pl


def kernel(x, mean, std, memory, W_enc, b_enc):
    raise NotImplementedError("write your pallas kernel here")


# speedup vs baseline: 1.1291x; 1.1291x over previous
import functools

import jax
import jax.numpy as jnp
from jax.experimental import pallas as pl
from jax.experimental.pallas import tpu as pltpu

MEM_LEN = 16384
OUT_DIM = 2048
IN_DIM = 1024
ROW_BLOCK = 1024


def _encoder_body(x_ref, mean_ref, std_ref, w_ref, b_ref, e_ref):
    x = x_ref[...]
    mean = mean_ref[...]
    std = std_ref[...]
    new = (x - mean) / (std + 1e-07)
    new = jnp.where(std == 0, jnp.zeros_like(new), new)
    logits = jnp.dot(new, w_ref[...], preferred_element_type=jnp.float32)
    logits = logits + b_ref[...]
    m = jnp.max(logits, axis=-1, keepdims=True)
    shifted = logits - m
    lse = jnp.log(jnp.sum(jnp.exp(shifted), axis=-1, keepdims=True))
    e_ref[...] = shifted - lse


def _scan_body(mem_ref, e_ref, out_ref, acc_ref):
    i = pl.program_id(0)

    @pl.when(i == 0)
    def _init():
        acc_ref[0, 0] = jnp.inf

    d = jnp.sum(jnp.abs(mem_ref[...] - e_ref[...]), axis=1)
    blk_min = jnp.min(d)
    acc_ref[0, 0] = jnp.minimum(acc_ref[0, 0], blk_min)

    @pl.when(i == pl.num_programs(0) - 1)
    def _emit():
        out_ref[0, 0] = acc_ref[0, 0]


@jax.jit
def kernel(x, mean, std, memory, W_enc, b_enc):
    mean2 = mean.reshape(1, IN_DIM)
    std2 = std.reshape(1, IN_DIM)
    b2 = b_enc.reshape(1, OUT_DIM)

    e = pl.pallas_call(
        _encoder_body,
        out_shape=jax.ShapeDtypeStruct((1, OUT_DIM), jnp.float32),
    )(x, mean2, std2, W_enc, b2)

    grid = MEM_LEN // ROW_BLOCK
    out = pl.pallas_call(
        _scan_body,
        grid=(grid,),
        in_specs=[
            pl.BlockSpec((ROW_BLOCK, OUT_DIM), lambda i: (i, 0)),
            pl.BlockSpec((1, OUT_DIM), lambda i: (0, 0)),
        ],
        out_specs=pl.BlockSpec(memory_space=pltpu.SMEM),
        out_shape=jax.ShapeDtypeStruct((1, 1), jnp.float32),
        scratch_shapes=[pltpu.SMEM((1, 1), jnp.float32)],
    )(memory, e)
    return out[0, 0]
